# Initial kernel scaffold; baseline (speedup 1.0000x reference)
import jax
import jax.numpy as jnp
from jax.experimental import pallas as pl


def kernel(x_user, x_wallet, edge_index_uw, edge_index_wu, edge_index_uu, edge_attr_uw, edge_attr_wu, edge_attr_uu, cate_scores, params):
    raise NotImplementedError("write your pallas kernel here")


# speedup vs baseline: 1.0767x; 1.0767x over previous
import functools

import jax
import jax.numpy as jnp
from jax import lax
from jax.experimental import pallas as pl
from jax.experimental.pallas import tpu as pltpu

NU = 10000
NW = 10000
E = 160000
HID = 64
HEADS = 4
OUT = 64


def _mm_body(x_ref, w_ref, b_ref, o_ref):
    o_ref[...] = (
        jnp.dot(x_ref[...], w_ref[...], preferred_element_type=jnp.float32)
        + b_ref[...]
    )


def _matmul_bias(x, w, b):
    n, k = x.shape
    m = w.shape[1]
    blk = 1000
    grid = n // blk
    return pl.pallas_call(
        _mm_body,
        grid=(grid,),
        in_specs=[
            pl.BlockSpec((blk, k), lambda i: (i, 0)),
            pl.BlockSpec((k, m), lambda i: (0, 0)),
            pl.BlockSpec((1, m), lambda i: (0, 0)),
        ],
        out_specs=pl.BlockSpec((blk, m), lambda i: (i, 0)),
        out_shape=jax.ShapeDtypeStruct((n, m), jnp.float32),
    )(x, w, b.reshape(1, -1))


def _gatv2(x_src, x_dst, edge_index, edge_attr, p, num_dst, heads, ch):
    src = edge_index[0]
    dst = edge_index[1]
    xl = x_src @ p["Wl"] + p["bl"]
    xr = x_dst @ p["Wr"] + p["br"]
    xj = xl[src].reshape(-1, heads, ch)
    xi = xr[dst].reshape(-1, heads, ch)
    x = xi + xj + (edge_attr @ p["We"]).reshape(-1, heads, ch)
    x = jax.nn.leaky_relu(x, negative_slope=0.2)
    alpha = jnp.sum(x * p["att"][None, :, :], axis=-1)
    ex = jnp.exp(alpha)
    denom = jax.ops.segment_sum(ex, dst, num_segments=num_dst)
    attn = ex / (denom[dst] + 1e-16)
    out = jax.ops.segment_sum(attn[:, :, None] * xj, dst, num_segments=num_dst)
    return out.reshape(num_dst, heads * ch) + p["bias"]


def kernel(x_user, x_wallet, edge_index_uw, edge_index_wu, edge_index_uu,
           edge_attr_uw, edge_attr_wu, edge_attr_uu, cate_scores, params):
    h_user = _matmul_bias(x_user, params["Win_user"], params["bin_user"])
    h_wallet = _matmul_bias(x_wallet, params["Win_wallet"], params["bin_wallet"])
    w1 = _gatv2(h_user, h_wallet, edge_index_uw, edge_attr_uw, params["l1_uw"], NW, HEADS, HID)
    u1 = _gatv2(h_wallet, h_user, edge_index_wu, edge_attr_wu, params["l1_wu"], NU, HEADS, HID) \
        + _gatv2(h_user, h_user, edge_index_uu, edge_attr_uu, params["l1_uu"], NU, HEADS, HID)
    u1 = jax.nn.relu(u1)
    w1 = jax.nn.relu(w1)
    w2 = _gatv2(u1, w1, edge_index_uw, edge_attr_uw, params["l2_uw"], NW, 1, HID)
    u2 = _gatv2(w1, u1, edge_index_wu, edge_attr_wu, params["l2_wu"], NU, 1, HID) \
        + _gatv2(u1, u1, edge_index_uu, edge_attr_uu, params["l2_uu"], NU, 1, HID)
    u2 = jax.nn.relu(u2)
    gate = jax.nn.sigmoid(cate_scores @ params["Wg"] + params["bg"])
    ue = u2 * gate
    return _matmul_bias(ue, params["Wout"], params["bout"])


# speedup vs baseline: 10.4410x; 9.6973x over previous
import functools

import jax
import jax.numpy as jnp
from jax import lax
from jax.experimental import pallas as pl
from jax.experimental.pallas import tpu as pltpu
from jax.experimental.pallas import tpu_sc as plsc

ND = 10000
E = 160000
NC = 2
NS = 16


def _mm_body(x_ref, w_ref, b_ref, o_ref):
    o_ref[...] = (
        jnp.dot(x_ref[...], w_ref[...], preferred_element_type=jnp.float32)
        + b_ref[...]
    )


def _matmul_bias(x, w, b):
    n, k = x.shape
    m = w.shape[1]
    blk = 1000
    return pl.pallas_call(
        _mm_body,
        grid=(n // blk,),
        in_specs=[
            pl.BlockSpec((blk, k), lambda i: (i, 0)),
            pl.BlockSpec((k, m), lambda i: (0, 0)),
            pl.BlockSpec((1, m), lambda i: (0, 0)),
        ],
        out_specs=pl.BlockSpec((blk, m), lambda i: (i, 0)),
        out_shape=jax.ShapeDtypeStruct((n, m), jnp.float32),
    )(x, w, b.reshape(1, -1))


def _final_body(u_ref, cate_ref, wg_ref, bg_ref, wo_ref, bo_ref, o_ref):
    gate = jax.nn.sigmoid(cate_ref[...] * wg_ref[...] + bg_ref[...])
    o_ref[...] = (
        jnp.dot(u_ref[...] * gate, wo_ref[...],
                preferred_element_type=jnp.float32)
        + bo_ref[...]
    )


def _final_proj(u2, cate, wg, bg, wo, bo):
    n, k = u2.shape
    m = wo.shape[1]
    blk = 1000
    return pl.pallas_call(
        _final_body,
        grid=(n // blk,),
        in_specs=[
            pl.BlockSpec((blk, k), lambda i: (i, 0)),
            pl.BlockSpec((blk, 1), lambda i: (i, 0)),
            pl.BlockSpec((1, k), lambda i: (0, 0)),
            pl.BlockSpec((1, k), lambda i: (0, 0)),
            pl.BlockSpec((k, m), lambda i: (0, 0)),
            pl.BlockSpec((1, m), lambda i: (0, 0)),
        ],
        out_specs=pl.BlockSpec((blk, m), lambda i: (i, 0)),
        out_shape=jax.ShapeDtypeStruct((n, m), jnp.float32),
    )(u2, cate, wg.reshape(1, -1), bg.reshape(1, -1), wo, bo.reshape(1, -1))


def _edge_assignment(headsplit, chunk, c, s):
    if headsplit:
        per_tile = E // NS
        base = s * per_tile
        trip = per_tile // chunk
        return trip, lambda k: base + k * chunk
    w = s * NC + c
    nchunks = E // chunk
    q, r = nchunks // (NC * NS), nchunks % (NC * NS)
    trip = jnp.where(w < r, q + 1, q)
    return trip, lambda k: (w + k * NC * NS) * chunk


def _make_edge_a(headsplit, wc, hc, chunk):
    nvh = wc // 16 // hc
    ztot = hc * ND
    zsl = ztot // 10
    zslp = (zsl + 15) // 16 * 16
    nco = NC if headsplit else 1
    mesh = plsc.VectorSubcoreMesh(core_axis_name="c", subcore_axis_name="s")
    out_type = (
        jax.ShapeDtypeStruct((nco * hc * E,), jnp.float32),
        jax.ShapeDtypeStruct((NC * ztot,), jnp.float32),
    )
    scratch = [
        pltpu.VMEM((chunk,), jnp.int32),
        pltpu.VMEM((chunk,), jnp.int32),
        pltpu.VMEM((chunk,), jnp.int32),
        pltpu.VMEM((chunk,), jnp.int32),
        pltpu.VMEM((chunk + 16,), jnp.float32),
        pltpu.VMEM((chunk, wc), jnp.float32),
        pltpu.VMEM((chunk, wc), jnp.float32),
        pltpu.VMEM((wc,), jnp.float32),
        pltpu.VMEM((wc,), jnp.float32),
        [pltpu.VMEM((chunk + 16,), jnp.float32) for _ in range(hc)],
        [pltpu.VMEM((chunk,), jnp.float32) for _ in range(hc)],
        pltpu.VMEM((zslp,), jnp.float32),
        pltpu.VMEM_SHARED((ztot,), jnp.float32),
        pltpu.SemaphoreType.DMA,
        pltpu.SemaphoreType.DMA,
    ]

    def body(tl, tr, srch, dsth, eah, wah, aah, exo, deno,
             srcv, dstv, idxv, idxd, eav, xlv, xrv, wev, attv, alph, exv,
             zb, dens, s1, s2):
        c = lax.axis_index("c")
        s = lax.axis_index("s")
        pltpu.sync_copy(wah.at[pl.ds(c * wc, wc)], wev)
        pltpu.sync_copy(aah.at[pl.ds(c * wc, wc)], attv)

        def zrow(i, carry):
            zb[pl.ds(i * 16, 16)] = jnp.zeros((16,), jnp.float32)
            return carry

        lax.fori_loop(0, zslp // 16, zrow, 0)

        @pl.when(s < 10)
        def _zero():
            pltpu.sync_copy(zb.at[pl.ds(0, zsl)],
                            dens.at[pl.ds(s * zsl, zsl)])

        plsc.subcore_barrier()
        trip, eb_fn = _edge_assignment(headsplit, chunk, c, s)

        def chunk_body(k, carry):
            eb = eb_fn(k)
            pltpu.sync_copy(srch.at[pl.ds(eb, chunk)], srcv)
            pltpu.sync_copy(dsth.at[pl.ds(eb, chunk)], dstv)
            pltpu.sync_copy(eah.at[pl.ds(eb, chunk)],
                            eav.at[pl.ds(0, chunk)])
            if headsplit:
                shift = c * ND
                for i in range(chunk // 16):
                    sl = pl.ds(i * 16, 16)
                    srcv[sl] = srcv[sl] + shift
                    idxv[sl] = dstv[sl] + shift
                gdst = idxv
            else:
                gdst = dstv
            cp1 = pltpu.async_copy(tl.at[srcv], xlv, s1)
            cp2 = pltpu.async_copy(tr.at[gdst], xrv, s2)
            cp1.wait()
            cp2.wait()

            def edge(i, carry2):
                eai = eav[pl.ds(i, 16)][0]
                for h in range(hc):
                    acc = None
                    for v in range(nvh):
                        cs = pl.ds((h * nvh + v) * 16, 16)
                        t = xlv[i, cs] + xrv[i, cs] + eai * wev[cs]
                        t = jnp.maximum(t, 0.2 * t)
                        term = attv[cs] * t
                        acc = term if acc is None else acc + term
                    alph[h][pl.ds(i, 16)] = jnp.broadcast_to(
                        jnp.sum(acc), (16,))
                return carry2

            lax.fori_loop(0, chunk, edge, 0)
            for h in range(hc):
                for j in range(chunk // 16):
                    sl = pl.ds(j * 16, 16)
                    exv[h][sl] = jnp.exp(alph[h][sl])
            for h in range(hc):
                if h == 0:
                    sidx = dstv
                else:
                    for i in range(chunk // 16):
                        sl = pl.ds(i * 16, 16)
                        idxd[sl] = dstv[sl] + h * ND
                    sidx = idxd
                pltpu.sync_copy(exv[h], dens.at[sidx], add=True)
                if headsplit:
                    exoff = (c * hc + h) * E + eb
                else:
                    exoff = h * E + eb
                pltpu.sync_copy(exv[h], exo.at[pl.ds(exoff, chunk)])
            return carry

        lax.fori_loop(0, trip, chunk_body, 0)
        plsc.subcore_barrier()

        @pl.when(s < 10)
        def _writeout():
            pltpu.sync_copy(dens.at[pl.ds(s * zsl, zsl)],
                            zb.at[pl.ds(0, zsl)])
            pltpu.sync_copy(zb.at[pl.ds(0, zsl)],
                            deno.at[pl.ds(c * ztot + s * zsl, zsl)])

    return pl.kernel(body, out_type=out_type, mesh=mesh,
                     scratch_types=scratch,
                     compiler_params=pltpu.CompilerParams(
                         needs_layout_passes=False,
                         use_tc_tiling_on_sc=False))


def _make_edge_b(headsplit, hc, chunk):
    wc = 64
    ztot = hc * ND
    rsl = (ND // NS) // 8 * 8
    br = rsl // 6
    mesh = plsc.VectorSubcoreMesh(core_axis_name="c", subcore_axis_name="s")
    out_type = jax.ShapeDtypeStruct((NC * hc * ND, wc), jnp.float32)
    scratch = [
        pltpu.VMEM((chunk,), jnp.int32),
        pltpu.VMEM((chunk,), jnp.int32),
        pltpu.VMEM((chunk,), jnp.float32),
        pltpu.VMEM((chunk + 16,), jnp.float32),
        pltpu.VMEM((chunk, wc), jnp.float32),
        pltpu.VMEM((chunk, wc), jnp.float32),
        pltpu.VMEM((ztot,), jnp.float32),
        pltpu.VMEM((br, wc), jnp.float32),
        pltpu.VMEM_SHARED((ND, wc), jnp.float32),
        pltpu.SemaphoreType.DMA,
    ]

    def body(tl, srch, dsth, exh, rdh, outo,
             srcv, dstv, exv, attnv, xlv, contrib, rdv, zb, outs, s1):
        c = lax.axis_index("c")
        s = lax.axis_index("s")
        if headsplit:
            pltpu.sync_copy(rdh.at[pl.ds(c * ztot, ztot)], rdv)
        else:
            pltpu.sync_copy(rdh, rdv)

        def zrow(i, carry):
            for j in range(wc // 16):
                zb[i, pl.ds(j * 16, 16)] = jnp.zeros((16,), jnp.float32)
            return carry

        trip, eb_fn = _edge_assignment(headsplit, chunk, c, s)
        tail = ND - NS * rsl

        for h in range(hc):
            lax.fori_loop(0, br, zrow, 0)
            for t in range(rsl // br):
                pltpu.sync_copy(zb, outs.at[pl.ds(s * rsl + t * br, br)])

            @pl.when(s == NS - 1)
            def _zero_tail():
                pltpu.sync_copy(zb.at[pl.ds(0, tail)],
                                outs.at[pl.ds(NS * rsl, tail)])

            plsc.subcore_barrier()

            def chunk_body(k, carry):
                eb = eb_fn(k)
                pltpu.sync_copy(srch.at[pl.ds(eb, chunk)], srcv)
                pltpu.sync_copy(dsth.at[pl.ds(eb, chunk)], dstv)
                if headsplit:
                    exoff = (c * hc + h) * E + eb
                else:
                    exoff = eb
                pltpu.sync_copy(exh.at[pl.ds(exoff, chunk)], exv)
                if headsplit:
                    roff = c * (2 * ND) + h
                    for i in range(chunk // 16):
                        sl = pl.ds(i * 16, 16)
                        srcv[sl] = srcv[sl] * 2 + roff
                cp = pltpu.async_copy(tl.at[srcv], xlv, s1)
                cp.wait()
                for i in range(chunk // 16):
                    sl = pl.ds(i * 16, 16)
                    idx = dstv[sl] + h * ND if h else dstv[sl]
                    rd = plsc.load_gather(rdv, [idx])
                    attnv[sl] = exv[sl] * rd

                def edge(i, carry2):
                    a = attnv[pl.ds(i, 16)][0]
                    for v in range(wc // 16):
                        cs = pl.ds(v * 16, 16)
                        contrib[i, cs] = a * xlv[i, cs]
                    return carry2

                lax.fori_loop(0, chunk, edge, 0)
                pltpu.sync_copy(contrib, outs.at[dstv], add=True)
                return carry

            lax.fori_loop(0, trip, chunk_body, 0)
            plsc.subcore_barrier()
            row0 = (c * hc + h) * ND
            for t in range(rsl // br):
                pltpu.sync_copy(outs.at[pl.ds(s * rsl + t * br, br)], zb)
                pltpu.sync_copy(zb, outo.at[pl.ds(row0 + s * rsl + t * br,
                                                  br)])

            @pl.when(s == NS - 1)
            def _write_tail():
                pltpu.sync_copy(outs.at[pl.ds(NS * rsl, tail)],
                                zb.at[pl.ds(0, tail)])
                pltpu.sync_copy(zb.at[pl.ds(0, tail)],
                                outo.at[pl.ds(row0 + NS * rsl, tail)])

            plsc.subcore_barrier()

    return pl.kernel(body, out_type=out_type, mesh=mesh,
                     scratch_types=scratch,
                     compiler_params=pltpu.CompilerParams(
                         needs_layout_passes=False,
                         use_tc_tiling_on_sc=False))


_edge_a_l1 = _make_edge_a(True, 128, 2, 80)
_edge_b_l1 = _make_edge_b(True, 2, 80)
_edge_a_l2 = _make_edge_a(False, 64, 1, 64)
_edge_b_l2 = _make_edge_b(False, 1, 64)


def _conv_l1(hs, hd, ei, ea, pp):
    xl = _matmul_bias(hs, pp["Wl"], pp["bl"])
    xr = _matmul_bias(hd, pp["Wr"], pp["br"])
    xl2 = xl.reshape(ND, 2, 128).swapaxes(0, 1).reshape(2 * ND, 128)
    xr2 = xr.reshape(ND, 2, 128).swapaxes(0, 1).reshape(2 * ND, 128)
    we2 = pp["We"].reshape(2, 128)
    att2 = pp["att"].reshape(2, 128)
    src, dst = ei[0], ei[1]
    ex, den = _edge_a_l1(xl2, xr2, src, dst, ea[:, 0],
                         we2.ravel(), att2.ravel())
    rden = 1.0 / (den + 1e-16)
    xl4 = xl2.reshape(4 * ND, 64)
    out = _edge_b_l1(xl4, src, dst, ex, rden)
    return (out.reshape(2, 2, ND, 64).transpose(2, 0, 1, 3).reshape(ND, 256)
            + pp["bias"])


def _conv_l2(hs, hd, ei, ea, pp):
    xl = _matmul_bias(hs, pp["Wl"], pp["bl"])
    xr = _matmul_bias(hd, pp["Wr"], pp["br"])
    we2 = jnp.broadcast_to(pp["We"].reshape(1, 64), (2, 64))
    att2 = jnp.broadcast_to(pp["att"].reshape(1, 64), (2, 64))
    src, dst = ei[0], ei[1]
    ex, den = _edge_a_l2(xl, xr, src, dst, ea[:, 0],
                         we2.ravel(), att2.ravel())
    den = den.reshape(2, ND)
    rden = 1.0 / (den[0] + den[1] + 1e-16)
    out = _edge_b_l2(xl, src, dst, ex, rden)
    out = out.reshape(2, ND, 64)
    return out[0] + out[1] + pp["bias"]


def kernel(x_user, x_wallet, edge_index_uw, edge_index_wu, edge_index_uu,
           edge_attr_uw, edge_attr_wu, edge_attr_uu, cate_scores, params):
    p = params
    hu = _matmul_bias(x_user, p["Win_user"], p["bin_user"])
    hw = _matmul_bias(x_wallet, p["Win_wallet"], p["bin_wallet"])
    w1 = jax.nn.relu(
        _conv_l1(hu, hw, edge_index_uw, edge_attr_uw, p["l1_uw"]))
    u1 = jax.nn.relu(
        _conv_l1(hw, hu, edge_index_wu, edge_attr_wu, p["l1_wu"])
        + _conv_l1(hu, hu, edge_index_uu, edge_attr_uu, p["l1_uu"]))
    u2 = jax.nn.relu(
        _conv_l2(w1, u1, edge_index_wu, edge_attr_wu, p["l2_wu"])
        + _conv_l2(u1, u1, edge_index_uu, edge_attr_uu, p["l2_uu"]))
    return _final_proj(u2, cate_scores, p["Wg"], p["bg"],
                       p["Wout"], p["bout"])
